# Initial kernel scaffold; baseline (speedup 1.0000x reference)
import jax
import jax.numpy as jnp
from jax.experimental import pallas as pl


def kernel(B, Bt):
    raise NotImplementedError("write your pallas kernel here")


# speedup vs baseline: 1.0289x; 1.0289x over previous
import jax
import jax.numpy as jnp
from jax.experimental import pallas as pl
from jax.experimental.pallas import tpu as pltpu

_SRC_X = 2048
_SRC_Y = 2048
_ROWS = 4096
_COLS = 4096
_BLK = 512


def _copy_scatter_kernel(bt_ref, b_ref, o_ref):
    i = pl.program_id(0)
    o_ref[...] = b_ref[...]

    @pl.when(i == _SRC_X // _BLK)
    def _():
        r = _SRC_X % _BLK
        row = b_ref[0, r : r + 1, :]
        col_ids = jax.lax.broadcasted_iota(jnp.int32, (1, _COLS), 1)
        o_ref[0, r : r + 1, :] = jnp.where(col_ids == _SRC_Y, bt_ref[0, 0], row)


def kernel(B, Bt):
    return pl.pallas_call(
        _copy_scatter_kernel,
        grid=(_ROWS // _BLK,),
        in_specs=[
            pl.BlockSpec(memory_space=pltpu.SMEM),
            pl.BlockSpec((1, _BLK, _COLS), lambda i: (0, i, 0)),
        ],
        out_specs=pl.BlockSpec((1, _BLK, _COLS), lambda i: (0, i, 0)),
        out_shape=jax.ShapeDtypeStruct((1, _ROWS, _COLS), jnp.float32),
    )(Bt, B)
